# baseline (speedup 1.0000x reference)
import functools
import jax
import jax.numpy as jnp
from jax import lax
from jax.experimental import pallas as pl
from jax.experimental.pallas import tpu as pltpu
from jax.experimental.pallas import tpu_sc as plsc

_info = plsc.get_sparse_core_info()
_NC, _NS = _info.num_cores, _info.num_subcores
_NW = _NC * _NS

_CHUNK = 16384


def _make_sc_add(n_total, n_emb):
    span = n_total // _NW
    n_chunks = span // _CHUNK

    @functools.partial(
        pl.kernel,
        mesh=plsc.VectorSubcoreMesh(core_axis_name="c", subcore_axis_name="s"),
        out_type=jax.ShapeDtypeStruct((n_total,), jnp.float32),
        scratch_types=[
            pltpu.VMEM((_CHUNK,), jnp.float32),
            pltpu.VMEM((_CHUNK,), jnp.float32),
        ],
    )
    def _sc_add(x_hbm, emb_hbm, out_hbm, xbuf, ebuf):
        wid = lax.axis_index("s") * _NC + lax.axis_index("c")
        base = wid * span

        def chunk_body(ci, carry):
            off = base + ci * _CHUNK
            eoff = lax.rem(off, n_emb)
            pltpu.sync_copy(x_hbm.at[pl.ds(off, _CHUNK)], xbuf)
            pltpu.sync_copy(emb_hbm.at[pl.ds(eoff, _CHUNK)], ebuf)

            def vec_body(i, c):
                s = pl.ds(i * 16, 16)
                xbuf[s] = xbuf[s] + ebuf[s]
                return c

            lax.fori_loop(0, _CHUNK // 16, vec_body, 0)
            pltpu.sync_copy(xbuf, out_hbm.at[pl.ds(off, _CHUNK)])
            return carry

        lax.fori_loop(0, n_chunks, chunk_body, 0)

    return _sc_add


def kernel(x, emb):
    batch, seq_len, d_model = x.shape
    n_total = batch * seq_len * d_model
    n_emb = seq_len * d_model
    out_flat = _make_sc_add(n_total, n_emb)(
        x.reshape(n_total), emb[:seq_len].reshape(n_emb)
    )
    return out_flat.reshape(batch, seq_len, d_model)


# speedup vs baseline: 8.5843x; 8.5843x over previous
import jax
import jax.numpy as jnp
from jax.experimental import pallas as pl


def _add_body(x_ref, emb_ref, out_ref):
    out_ref[0] = x_ref[0] + emb_ref[...]


def kernel(x, emb):
    batch, seq_len, d_model = x.shape
    blk = 2048
    while seq_len % blk:
        blk //= 2
    n_seq = seq_len // blk
    return pl.pallas_call(
        _add_body,
        grid=(n_seq, batch),
        in_specs=[
            pl.BlockSpec((1, blk, d_model), lambda s, b: (b, s, 0)),
            pl.BlockSpec((blk, d_model), lambda s, b: (s, 0)),
        ],
        out_specs=pl.BlockSpec((1, blk, d_model), lambda s, b: (b, s, 0)),
        out_shape=jax.ShapeDtypeStruct((batch, seq_len, d_model), x.dtype),
    )(x, emb)


# speedup vs baseline: 8.5942x; 1.0012x over previous
import jax
from jax.experimental import pallas as pl


def _add_body(x_ref, emb_ref, out_ref):
    out_ref[0] = x_ref[0] + emb_ref[...]


def kernel(x, emb):
    batch, seq_len, d_model = x.shape
    blk = 2048
    while seq_len % blk:
        blk //= 2
    n_seq = seq_len // blk
    return pl.pallas_call(
        _add_body,
        grid=(n_seq, batch),
        in_specs=[
            pl.BlockSpec((1, blk, d_model), lambda s, b: (b, s, 0)),
            pl.BlockSpec((blk, d_model), lambda s, b: (s, 0)),
        ],
        out_specs=pl.BlockSpec((1, blk, d_model), lambda s, b: (b, s, 0)),
        out_shape=jax.ShapeDtypeStruct((batch, seq_len, d_model), x.dtype),
    )(x, emb)
